# Initial kernel scaffold; baseline (speedup 1.0000x reference)
import functools

import jax
import jax.numpy as jnp
from jax import lax
from jax.experimental import pallas as pl
from jax.experimental.pallas import tpu as pltpu
from jax.experimental.pallas import tpu_sc as plsc

N = 10000
E = 320000
F_IN = 128
HID = 8
HEADS = 8
NCLS = 16

NCORES = 2
NSUB = 16
NW = NCORES * NSUB
CB = 128
NCHUNK = E // CB
NT = -(-NCHUNK // NW)
RCH = 125
NRCH = N // RCH
W1COLS = 80
W2COLS = 32

_mesh = plsc.VectorSubcoreMesh(
    core_axis_name="c", subcore_axis_name="s",
    num_cores=NCORES, num_subcores=NSUB)


def _tc_a_body(x_ref, w1_ref, asm_ref, adm_ref, tsrc_ref, adt_ref, gmax_ref):
    h = jnp.dot(x_ref[...], w1_ref[...], preferred_element_type=jnp.float32)
    a_s = jnp.dot(h, asm_ref[...], preferred_element_type=jnp.float32)
    a_d = jnp.dot(h, adm_ref[...], preferred_element_type=jnp.float32)
    z8 = jnp.zeros((N, HEADS), jnp.float32)
    tsrc_ref[...] = jnp.concatenate([h, a_s, z8], axis=1)
    adt_ref[...] = a_d
    g = jnp.max(a_s, axis=0, keepdims=True)
    gmax_ref[...] = jnp.concatenate([g, jnp.zeros((1, 8), jnp.float32)], axis=1)


def _tc_a(x, w1, asm, adm):
    return pl.pallas_call(
        _tc_a_body,
        out_shape=[
            jax.ShapeDtypeStruct((N, W1COLS), jnp.float32),
            jax.ShapeDtypeStruct((N, HEADS), jnp.float32),
            jax.ShapeDtypeStruct((1, 16), jnp.float32),
        ],
    )(x, w1, asm, adm)


def _tc_b_body(accp_ref, b1_ref, w2_ref, as2_ref, ad2_ref,
               tsrc2_ref, adt2_ref, gmax2_ref):
    a0 = accp_ref[0]
    a1 = accp_ref[1]
    num = a0[:, 0:64] + a1[:, 0:64]
    den = a0[:, 64:72] + a1[:, 64:72]
    cols = []
    for h in range(HEADS):
        cols.append(num[:, 8 * h:8 * h + 8] / (den[:, h:h + 1] + 1e-16))
    o = jnp.concatenate(cols, axis=1) + b1_ref[...]
    o = jnp.where(o > 0, o, jnp.exp(jnp.minimum(o, 0.0)) - 1.0)
    h2 = jnp.dot(o, w2_ref[...], preferred_element_type=jnp.float32)
    a_s = jnp.sum(h2 * as2_ref[...], axis=1, keepdims=True)
    a_d = jnp.sum(h2 * ad2_ref[...], axis=1, keepdims=True)
    z15 = jnp.zeros((N, 15), jnp.float32)
    tsrc2_ref[...] = jnp.concatenate([h2, a_s, z15], axis=1)
    adt2_ref[...] = a_d
    g = jnp.max(a_s, axis=0, keepdims=True)
    gmax2_ref[...] = jnp.concatenate([g, jnp.zeros((1, 15), jnp.float32)], axis=1)


def _tc_b(accp, b1r, w2, as2, ad2):
    return pl.pallas_call(
        _tc_b_body,
        out_shape=[
            jax.ShapeDtypeStruct((N, W2COLS), jnp.float32),
            jax.ShapeDtypeStruct((N, 1), jnp.float32),
            jax.ShapeDtypeStruct((1, 16), jnp.float32),
        ],
    )(accp, b1r, w2, as2, ad2)


def _tc_c_body(accp_ref, b2_ref, out_ref):
    a0 = accp_ref[0]
    a1 = accp_ref[1]
    num = a0[:, 0:16] + a1[:, 0:16]
    den = a0[:, 16:17] + a1[:, 16:17]
    out_ref[...] = num / (den + 1e-16) + b2_ref[...]


def _tc_c(accp, b2r):
    return pl.pallas_call(
        _tc_c_body,
        out_shape=jax.ShapeDtypeStruct((N, NCLS), jnp.float32),
    )(accp, b2r)


@functools.partial(
    pl.kernel,
    out_type=jax.ShapeDtypeStruct((NCORES, N, W1COLS), jnp.float32),
    mesh=_mesh,
    scratch_types=[
        pltpu.VMEM((CB, W1COLS), jnp.float32),
        pltpu.VMEM((CB, W1COLS), jnp.float32),
        pltpu.VMEM((N, HEADS), jnp.float32),
        pltpu.VMEM((CB,), jnp.int32),
        pltpu.VMEM((CB,), jnp.int32),
        pltpu.SMEM((16,), jnp.float32),
        pltpu.VMEM_SHARED((N, W1COLS), jnp.float32),
        pltpu.SemaphoreType.DMA,
    ],
)
def _sc1(edge_hbm, tsrc_hbm, adt_hbm, gmax_hbm, zrows_hbm, accp_hbm,
         s_v, o_v, ad_v, isrc_v, idst_v, gmax_s, acc_sh, sem):
    cidx = lax.axis_index("c")
    sid = lax.axis_index("s")
    wid = sid * NCORES + cidx

    pltpu.sync_copy(gmax_hbm.at[0], gmax_s)
    pltpu.sync_copy(adt_hbm, ad_v)
    pltpu.sync_copy(zrows_hbm, o_v)

    def zbody(k, carry):
        ch = sid + k * NSUB
        pltpu.sync_copy(zrows_hbm.at[pl.ds(0, RCH)],
                        acc_sh.at[pl.ds(ch * RCH, RCH)])
        return carry
    lax.fori_loop(0, NRCH // NSUB, zbody, 0)
    plsc.subcore_barrier()

    iota16 = lax.iota(jnp.int32, 16)

    def chunk_body(t, carry):
        c = wid + t * NW

        @pl.when(c < NCHUNK)
        def _():
            base = c * CB
            pltpu.sync_copy(edge_hbm.at[0, pl.ds(base, CB)], isrc_v)
            pltpu.sync_copy(edge_hbm.at[1, pl.ds(base, CB)], idst_v)
            pltpu.async_copy(tsrc_hbm.at[isrc_v], s_v, sem).wait()

            def ibody(i, icarry):
                rows = i * 16 + iota16
                vdst = plsc.load_gather(idst_v, [rows])
                for h in range(HEADS):
                    colh = jnp.full((16,), h, jnp.int32)
                    ad_h = plsc.load_gather(ad_v, [vdst, colh])
                    as_h = plsc.load_gather(s_v, [rows, colh + 64])
                    z = as_h + ad_h
                    z = jnp.where(z > 0, z, 0.2 * z)
                    bound = jnp.maximum(gmax_s[h] + ad_h, 0.0)
                    ex = jnp.exp(z - bound)
                    plsc.store_scatter(o_v, [rows, colh + 64], ex)
                    for cc in range(HID):
                        col = jnp.full((16,), 8 * h + cc, jnp.int32)
                        v = plsc.load_gather(s_v, [rows, col]) * ex
                        plsc.store_scatter(o_v, [rows, col], v)
                return icarry
            lax.fori_loop(0, CB // 16, ibody, 0)
            pltpu.sync_copy(o_v, acc_sh.at[idst_v], add=True)
        return carry
    lax.fori_loop(0, NT, chunk_body, 0)
    plsc.subcore_barrier()

    def cbody(k, carry):
        ch = sid + k * NSUB
        pltpu.sync_copy(acc_sh.at[pl.ds(ch * RCH, RCH)],
                        accp_hbm.at[cidx, pl.ds(ch * RCH, RCH)])
        return carry
    lax.fori_loop(0, NRCH // NSUB, cbody, 0)


@functools.partial(
    pl.kernel,
    out_type=jax.ShapeDtypeStruct((NCORES, N, W2COLS), jnp.float32),
    mesh=_mesh,
    scratch_types=[
        pltpu.VMEM((CB, W2COLS), jnp.float32),
        pltpu.VMEM((CB, W2COLS), jnp.float32),
        pltpu.VMEM((N, 1), jnp.float32),
        pltpu.VMEM((CB,), jnp.int32),
        pltpu.VMEM((CB,), jnp.int32),
        pltpu.SMEM((16,), jnp.float32),
        pltpu.VMEM_SHARED((N, W2COLS), jnp.float32),
        pltpu.SemaphoreType.DMA,
    ],
)
def _sc2(edge_hbm, tsrc_hbm, adt_hbm, gmax_hbm, zrows_hbm, accp_hbm,
         s_v, o_v, ad_v, isrc_v, idst_v, gmax_s, acc_sh, sem):
    cidx = lax.axis_index("c")
    sid = lax.axis_index("s")
    wid = sid * NCORES + cidx

    pltpu.sync_copy(gmax_hbm.at[0], gmax_s)
    pltpu.sync_copy(adt_hbm, ad_v)
    pltpu.sync_copy(zrows_hbm, o_v)

    def zbody(k, carry):
        ch = sid + k * NSUB
        pltpu.sync_copy(zrows_hbm.at[pl.ds(0, RCH)],
                        acc_sh.at[pl.ds(ch * RCH, RCH)])
        return carry
    lax.fori_loop(0, NRCH // NSUB, zbody, 0)
    plsc.subcore_barrier()

    iota16 = lax.iota(jnp.int32, 16)
    zero16 = jnp.zeros((16,), jnp.int32)

    def chunk_body(t, carry):
        c = wid + t * NW

        @pl.when(c < NCHUNK)
        def _():
            base = c * CB
            pltpu.sync_copy(edge_hbm.at[0, pl.ds(base, CB)], isrc_v)
            pltpu.sync_copy(edge_hbm.at[1, pl.ds(base, CB)], idst_v)
            pltpu.async_copy(tsrc_hbm.at[isrc_v], s_v, sem).wait()

            def ibody(i, icarry):
                rows = i * 16 + iota16
                vdst = plsc.load_gather(idst_v, [rows])
                col16 = jnp.full((16,), 16, jnp.int32)
                ad = plsc.load_gather(ad_v, [vdst, zero16])
                a_s = plsc.load_gather(s_v, [rows, col16])
                z = a_s + ad
                z = jnp.where(z > 0, z, 0.2 * z)
                bound = jnp.maximum(gmax_s[0] + ad, 0.0)
                ex = jnp.exp(z - bound)
                plsc.store_scatter(o_v, [rows, col16], ex)
                for cc in range(NCLS):
                    col = jnp.full((16,), cc, jnp.int32)
                    v = plsc.load_gather(s_v, [rows, col]) * ex
                    plsc.store_scatter(o_v, [rows, col], v)
                return icarry
            lax.fori_loop(0, CB // 16, ibody, 0)
            pltpu.sync_copy(o_v, acc_sh.at[idst_v], add=True)
        return carry
    lax.fori_loop(0, NT, chunk_body, 0)
    plsc.subcore_barrier()

    def cbody(k, carry):
        ch = sid + k * NSUB
        pltpu.sync_copy(acc_sh.at[pl.ds(ch * RCH, RCH)],
                        accp_hbm.at[cidx, pl.ds(ch * RCH, RCH)])
        return carry
    lax.fori_loop(0, NRCH // NSUB, cbody, 0)


def kernel(x, edge_index, W1, att_src1, att_dst1, b1, W2, att_src2, att_dst2, b2):
    eye8 = jnp.eye(HEADS, dtype=jnp.float32)
    asm = (att_src1[:, :, None] * eye8[:, None, :]).reshape(HEADS * HID, HEADS)
    adm = (att_dst1[:, :, None] * eye8[:, None, :]).reshape(HEADS * HID, HEADS)
    zrows1 = jnp.zeros((CB, W1COLS), jnp.float32)
    zrows2 = jnp.zeros((CB, W2COLS), jnp.float32)

    tsrc, adt, gmax1 = _tc_a(x, W1, asm, adm)
    accp1 = _sc1(edge_index, tsrc, adt, gmax1, zrows1)
    tsrc2, adt2, gmax2 = _tc_b(accp1, b1.reshape(1, 64), W2, att_src2, att_dst2)
    accp2 = _sc2(edge_index, tsrc2, adt2, gmax2, zrows2)
    return _tc_c(accp2, b2.reshape(1, 16))


# speedup vs baseline: 34.1043x; 34.1043x over previous
import functools

import jax
import jax.numpy as jnp
from jax import lax
from jax.experimental import pallas as pl
from jax.experimental.pallas import tpu as pltpu
from jax.experimental.pallas import tpu_sc as plsc

N = 10000
E = 320000
F_IN = 128
HID = 8
HEADS = 8
NCLS = 16

NCORES = 2
NSUB = 16
NW = NCORES * NSUB
NGRP = 313
EPAD = NGRP * 1024
NBLK = -(-NGRP // NW)
RCH = 200
NRCH = N // RCH
NZT = -(-NRCH // NSUB)
W1COLS = 128
W2COLS = 128

_mesh = plsc.VectorSubcoreMesh(
    core_axis_name="c", subcore_axis_name="s",
    num_cores=NCORES, num_subcores=NSUB)


def _tc_a_body(x_ref, w1_ref, asm_ref, adm_ref, tsrc_ref, adt_ref, gmax_ref):
    h = jnp.dot(x_ref[...], w1_ref[...], preferred_element_type=jnp.float32)
    a_s = jnp.dot(h, asm_ref[...], preferred_element_type=jnp.float32)
    a_d = jnp.dot(h, adm_ref[...], preferred_element_type=jnp.float32)
    zpad = jnp.zeros((N, W1COLS - 72), jnp.float32)
    tsrc_ref[...] = jnp.concatenate([h, a_s, zpad], axis=1)
    adt_ref[...] = a_d
    g8 = jnp.max(a_s, axis=0)
    gmax_ref[...] = jnp.broadcast_to(g8[:, None], (8, 128))


def _tc_a(x, w1, asm, adm):
    return pl.pallas_call(
        _tc_a_body,
        out_shape=[
            jax.ShapeDtypeStruct((N, W1COLS), jnp.float32),
            jax.ShapeDtypeStruct((N, HEADS), jnp.float32),
            jax.ShapeDtypeStruct((8, 128), jnp.float32),
        ],
    )(x, w1, asm, adm)


def _tc_b_body(accp_ref, b1_ref, w2_ref, as2_ref, ad2_ref,
               tsrc2_ref, adt2_ref, gmax2_ref):
    a0 = accp_ref[0]
    a1 = accp_ref[1]
    num = a0[:, 0:64] + a1[:, 0:64]
    den = a0[:, 64:72] + a1[:, 64:72]
    cols = []
    for h in range(HEADS):
        cols.append(num[:, 8 * h:8 * h + 8] / (den[:, h:h + 1] + 1e-16))
    o = jnp.concatenate(cols, axis=1) + b1_ref[...]
    o = jnp.where(o > 0, o, jnp.exp(jnp.minimum(o, 0.0)) - 1.0)
    h2 = jnp.dot(o, w2_ref[...], preferred_element_type=jnp.float32)
    a_s = jnp.sum(h2 * as2_ref[...], axis=1, keepdims=True)
    a_d = jnp.sum(h2 * ad2_ref[...], axis=1, keepdims=True)
    zpad = jnp.zeros((N, W2COLS - 17), jnp.float32)
    tsrc2_ref[...] = jnp.concatenate([h2, a_s, zpad], axis=1)
    adt2_ref[...] = a_d
    gmax2_ref[...] = jnp.broadcast_to(jnp.max(a_s), (8, 128))


def _tc_b(accp, b1r, w2, as2, ad2):
    return pl.pallas_call(
        _tc_b_body,
        out_shape=[
            jax.ShapeDtypeStruct((N, W2COLS), jnp.float32),
            jax.ShapeDtypeStruct((N, 1), jnp.float32),
            jax.ShapeDtypeStruct((8, 128), jnp.float32),
        ],
    )(accp, b1r, w2, as2, ad2)


def _tc_c_body(accp_ref, b2_ref, out_ref):
    a0 = accp_ref[0]
    a1 = accp_ref[1]
    num = a0[:, 0:16] + a1[:, 0:16]
    den = a0[:, 16:17] + a1[:, 16:17]
    out_ref[...] = num / (den + 1e-16) + b2_ref[...]


def _tc_c(accp, b2r):
    return pl.pallas_call(
        _tc_c_body,
        out_shape=jax.ShapeDtypeStruct((N, NCLS), jnp.float32),
    )(accp, b2r)


@functools.partial(
    pl.kernel,
    out_type=jax.ShapeDtypeStruct((NCORES, N, W1COLS), jnp.float32),
    mesh=_mesh,
    compiler_params=pltpu.CompilerParams(needs_layout_passes=False),
    scratch_types=[
        pltpu.VMEM((64, 128), jnp.float32),
        pltpu.VMEM((64, 128), jnp.float32),
        pltpu.VMEM((64, 128), jnp.float32),
        pltpu.VMEM((64, 128), jnp.float32),
        pltpu.VMEM((16, 64), jnp.int32),
        pltpu.VMEM((16, 64), jnp.int32),
        pltpu.VMEM((16, 64), jnp.int32),
        pltpu.VMEM((8, 128), jnp.float32),
        pltpu.VMEM_SHARED((N // 16 + 7, 128), jnp.float32),
        pltpu.VMEM_SHARED((N + 16, W1COLS), jnp.float32),
        pltpu.SemaphoreType.DMA,
        pltpu.SemaphoreType.DMA,
        pltpu.SemaphoreType.DMA,
        pltpu.SemaphoreType.DMA,
    ],
)
def _sc1(src_hbm, dst_hbm, tsrc_hbm, adt_hbm, gmax_hbm, zrows_hbm, accp_hbm,
         s_a, s_b, d_a, d_b, srcb, dstb, irowb, gmax_v, adt_sh, acc_sh,
         sem_sa, sem_sb, sem_da, sem_db):
    cidx = lax.axis_index("c")
    sid = lax.axis_index("s")
    wid = sid * NCORES + cidx

    pltpu.sync_copy(gmax_hbm, gmax_v)

    @pl.when(sid == 0)
    def _():
        pltpu.sync_copy(adt_hbm, adt_sh)

    def zbody(k, carry):
        ch = sid + k * NSUB

        @pl.when(ch < NRCH)
        def _():
            pltpu.sync_copy(zrows_hbm.at[pl.ds(0, RCH)],
                            acc_sh.at[pl.ds(ch * RCH, RCH)])
        return carry
    lax.fori_loop(0, NZT, zbody, 0)
    plsc.subcore_barrier()

    iota16 = lax.iota(jnp.int32, 16)
    gvec = [gmax_v[h, pl.ds(0, 16)] for h in range(HEADS)]

    def fire_s(hc, buf, sem):
        pltpu.async_copy(tsrc_hbm.at[srcb.at[hc]], buf, sem)

    def fire_d(hc, buf, sem):
        pltpu.async_copy(adt_sh.at[irowb.at[hc]], buf, sem)

    def wait_s(hc, buf, sem):
        pltpu.make_async_copy(tsrc_hbm.at[srcb.at[hc]], buf, sem).wait()

    def wait_d(hc, buf, sem):
        pltpu.make_async_copy(adt_sh.at[irowb.at[hc]], buf, sem).wait()

    def compute_half(hc, s_v, d_v):
        def ibody(i, icarry):
            rows = i * 16 + iota16
            vdst = plsc.load_gather(
                dstb, [jnp.full((16,), hc, jnp.int32), rows])
            adcol = jnp.left_shift(jnp.bitwise_and(vdst, 15), 3)
            for h in range(HEADS):
                ad_h = plsc.load_gather(d_v, [rows, adcol + h])
                as_h = plsc.load_gather(
                    s_v, [rows, jnp.full((16,), 64 + h, jnp.int32)])
                z = as_h + ad_h
                z = jnp.where(z > 0, z, 0.2 * z)
                bound = jnp.maximum(gvec[h] + ad_h, 0.0)
                ex = jnp.exp(z - bound)
                plsc.store_scatter(
                    s_v, [rows, jnp.full((16,), 64 + h, jnp.int32)], ex)
                for cc in range(HID):
                    col = jnp.full((16,), 8 * h + cc, jnp.int32)
                    v = plsc.load_gather(s_v, [rows, col]) * ex
                    plsc.store_scatter(s_v, [rows, col], v)
            return icarry
        lax.fori_loop(0, 4, ibody, 0)

    def blk(b, carry):
        gid = wid + b * NW

        @pl.when(gid < NGRP)
        def _():
            pltpu.sync_copy(src_hbm.at[pl.ds(gid * 16, 16)], srcb)
            pltpu.sync_copy(dst_hbm.at[pl.ds(gid * 16, 16)], dstb)

            def rbody(q, rcarry):
                ridx = [jnp.full((16,), jnp.right_shift(q, 2), jnp.int32),
                        jnp.left_shift(jnp.bitwise_and(q, 3), 4) + iota16]
                vd = plsc.load_gather(dstb, ridx)
                plsc.store_scatter(irowb, ridx, jnp.right_shift(vd, 4))
                return rcarry
            lax.fori_loop(0, 64, rbody, 0)

            fire_s(0, s_a, sem_sa)
            fire_d(0, d_a, sem_da)
            fire_s(1, s_b, sem_sb)
            fire_d(1, d_b, sem_db)

            def jbody(j, jcarry):
                hca = 2 * j
                hcb = 2 * j + 1
                wait_s(hca, s_a, sem_sa)
                wait_d(hca, d_a, sem_da)
                compute_half(hca, s_a, d_a)
                pltpu.sync_copy(s_a, acc_sh.at[dstb.at[hca]], add=True)

                @pl.when(j < 7)
                def _():
                    fire_s(hca + 2, s_a, sem_sa)
                    fire_d(hca + 2, d_a, sem_da)
                wait_s(hcb, s_b, sem_sb)
                wait_d(hcb, d_b, sem_db)
                compute_half(hcb, s_b, d_b)
                pltpu.sync_copy(s_b, acc_sh.at[dstb.at[hcb]], add=True)

                @pl.when(j < 7)
                def _():
                    fire_s(hcb + 2, s_b, sem_sb)
                    fire_d(hcb + 2, d_b, sem_db)
                return jcarry
            lax.fori_loop(0, 8, jbody, 0)
        return carry
    lax.fori_loop(0, NBLK, blk, 0)
    plsc.subcore_barrier()

    def cbody(k, carry):
        ch = sid + k * NSUB

        @pl.when(ch < NRCH)
        def _():
            pltpu.sync_copy(acc_sh.at[pl.ds(ch * RCH, RCH)],
                            accp_hbm.at[cidx, pl.ds(ch * RCH, RCH)])
        return carry
    lax.fori_loop(0, NZT, cbody, 0)


@functools.partial(
    pl.kernel,
    out_type=jax.ShapeDtypeStruct((NCORES, N, W2COLS), jnp.float32),
    mesh=_mesh,
    compiler_params=pltpu.CompilerParams(needs_layout_passes=False),
    scratch_types=[
        pltpu.VMEM((64, 128), jnp.float32),
        pltpu.VMEM((64, 128), jnp.float32),
        pltpu.VMEM((80, 128), jnp.float32),
        pltpu.VMEM((16, 64), jnp.int32),
        pltpu.VMEM((16, 64), jnp.int32),
        pltpu.VMEM((8, 128), jnp.float32),
        pltpu.VMEM_SHARED((N + 16, W2COLS), jnp.float32),
        pltpu.SemaphoreType.DMA,
        pltpu.SemaphoreType.DMA,
    ],
)
def _sc2(src_hbm, dst_hbm, tsrc_hbm, adt_hbm, gmax_hbm, zrows_hbm, accp_hbm,
         s_a, s_b, ad_v, srcb, dstb, gmax_v, acc_sh, sem_sa, sem_sb):
    cidx = lax.axis_index("c")
    sid = lax.axis_index("s")
    wid = sid * NCORES + cidx

    pltpu.sync_copy(gmax_hbm, gmax_v)
    pltpu.sync_copy(adt_hbm, ad_v)

    def zbody(k, carry):
        ch = sid + k * NSUB

        @pl.when(ch < NRCH)
        def _():
            pltpu.sync_copy(zrows_hbm.at[pl.ds(0, RCH)],
                            acc_sh.at[pl.ds(ch * RCH, RCH)])
        return carry
    lax.fori_loop(0, NZT, zbody, 0)
    plsc.subcore_barrier()

    iota16 = lax.iota(jnp.int32, 16)
    gvec0 = gmax_v[0, pl.ds(0, 16)]

    def fire_s(hc, buf, sem):
        pltpu.async_copy(tsrc_hbm.at[srcb.at[hc]], buf, sem)

    def wait_s(hc, buf, sem):
        pltpu.make_async_copy(tsrc_hbm.at[srcb.at[hc]], buf, sem).wait()

    def compute_half(hc, s_v):
        def ibody(i, icarry):
            rows = i * 16 + iota16
            vdst = plsc.load_gather(
                dstb, [jnp.full((16,), hc, jnp.int32), rows])
            col16 = jnp.full((16,), 16, jnp.int32)
            ad = plsc.load_gather(
                ad_v, [jnp.right_shift(vdst, 7), jnp.bitwise_and(vdst, 127)])
            a_s = plsc.load_gather(s_v, [rows, col16])
            z = a_s + ad
            z = jnp.where(z > 0, z, 0.2 * z)
            bound = jnp.maximum(gvec0 + ad, 0.0)
            ex = jnp.exp(z - bound)
            plsc.store_scatter(s_v, [rows, col16], ex)
            for cc in range(NCLS):
                col = jnp.full((16,), cc, jnp.int32)
                v = plsc.load_gather(s_v, [rows, col]) * ex
                plsc.store_scatter(s_v, [rows, col], v)
            return icarry
        lax.fori_loop(0, 4, ibody, 0)

    def blk(b, carry):
        gid = wid + b * NW

        @pl.when(gid < NGRP)
        def _():
            pltpu.sync_copy(src_hbm.at[pl.ds(gid * 16, 16)], srcb)
            pltpu.sync_copy(dst_hbm.at[pl.ds(gid * 16, 16)], dstb)
            fire_s(0, s_a, sem_sa)
            fire_s(1, s_b, sem_sb)

            def jbody(j, jcarry):
                hca = 2 * j
                hcb = 2 * j + 1
                wait_s(hca, s_a, sem_sa)
                compute_half(hca, s_a)
                pltpu.sync_copy(s_a, acc_sh.at[dstb.at[hca]], add=True)

                @pl.when(j < 7)
                def _():
                    fire_s(hca + 2, s_a, sem_sa)
                wait_s(hcb, s_b, sem_sb)
                compute_half(hcb, s_b)
                pltpu.sync_copy(s_b, acc_sh.at[dstb.at[hcb]], add=True)

                @pl.when(j < 7)
                def _():
                    fire_s(hcb + 2, s_b, sem_sb)
                return jcarry
            lax.fori_loop(0, 8, jbody, 0)
        return carry
    lax.fori_loop(0, NBLK, blk, 0)
    plsc.subcore_barrier()

    def cbody(k, carry):
        ch = sid + k * NSUB

        @pl.when(ch < NRCH)
        def _():
            pltpu.sync_copy(acc_sh.at[pl.ds(ch * RCH, RCH)],
                            accp_hbm.at[cidx, pl.ds(ch * RCH, RCH)])
        return carry
    lax.fori_loop(0, NZT, cbody, 0)


def kernel(x, edge_index, W1, att_src1, att_dst1, b1, W2, att_src2, att_dst2, b2):
    eye8 = jnp.eye(HEADS, dtype=jnp.float32)
    asm = (att_src1[:, :, None] * eye8[:, None, :]).reshape(HEADS * HID, HEADS)
    adm = (att_dst1[:, :, None] * eye8[:, None, :]).reshape(HEADS * HID, HEADS)
    zrows = jnp.zeros((RCH, 128), jnp.float32)

    src_e = edge_index[0]
    dst_e = edge_index[1]
    srcp = jnp.concatenate(
        [src_e, jnp.zeros((EPAD - E,), jnp.int32)]).reshape(NGRP * 16, 64)
    dstp = jnp.concatenate(
        [dst_e, jnp.full((EPAD - E,), N, jnp.int32)]).reshape(NGRP * 16, 64)

    tsrc, adt, gmax1 = _tc_a(x, W1, asm, adm)
    adt1p = jnp.concatenate(
        [adt.reshape(N // 16, 128), jnp.zeros((7, 128), jnp.float32)])
    accp1 = _sc1(srcp, dstp, tsrc, adt1p, gmax1, zrows)
    tsrc2, adt2, gmax2 = _tc_b(accp1, b1.reshape(1, 64), W2, att_src2, att_dst2)
    adt2p = jnp.concatenate(
        [adt2.reshape(N), jnp.zeros((240,), jnp.float32)]).reshape(80, 128)
    accp2 = _sc2(srcp, dstp, tsrc2, adt2p, gmax2, zrows)
    return _tc_c(accp2, b2.reshape(1, 16))
